# Initial kernel scaffold; baseline (speedup 1.0000x reference)
import jax
import jax.numpy as jnp
from jax.experimental import pallas as pl


def kernel(x, edge_index, Wq, bq, Wk, bk, Wv, bv, Wskip, bskip, ln1_g, ln1_b, W1, b1, W2, b2, ln2_g, ln2_b):
    raise NotImplementedError("write your pallas kernel here")


# speedup vs baseline: 7.1834x; 7.1834x over previous
import jax
import jax.numpy as jnp
from jax import lax
from jax.experimental import pallas as pl
from jax.experimental.pallas import tpu as pltpu
from jax.experimental.pallas import tpu_sc as plsc

N = 10000
E = 160000
D = 128
H = 8
C = 128
FF = 4 * D

NPAD = 10240
NSC = 10112
BLK = 64
EPT = 10048
EPAD = EPT * 16
AROWS = NSC // 16
DROWS = NPAD // 16
NBLK = EPT // BLK
INV_SQRT_C = 1.0 / (C ** 0.5)

_GDN = lax.GatherDimensionNumbers(
    offset_dims=(), collapsed_slice_dims=(0,), start_index_map=(0,))


def _shuf(x, idx):
    return lax.gather(x, idx.reshape(16, 1), _GDN, (1,),
                      mode=lax.GatherScatterMode.PROMISE_IN_BOUNDS)


def _qkv_body(x_ref, wq_ref, wk_ref, wv_ref, bq_ref, bk_ref, bv_ref,
              q_ref, k_ref, v_ref):
    xb = x_ref[...]
    q_ref[0] = jnp.dot(xb, wq_ref[...], preferred_element_type=jnp.float32) + bq_ref[0, 0]
    k_ref[0] = jnp.dot(xb, wk_ref[...], preferred_element_type=jnp.float32) + bk_ref[0, 0]
    v_ref[0] = jnp.dot(xb, wv_ref[...], preferred_element_type=jnp.float32) + bv_ref[0, 0]


def _qkv(x_pad, Wq, Wk, Wv, bq, bk, bv):
    RB = 512
    grid = (NPAD // RB, H)
    out_sh = jax.ShapeDtypeStruct((H, NPAD, C), jnp.float32)
    return pl.pallas_call(
        _qkv_body,
        grid=grid,
        in_specs=[
            pl.BlockSpec((RB, D), lambda i, h: (i, 0)),
            pl.BlockSpec((D, C), lambda i, h: (0, h)),
            pl.BlockSpec((D, C), lambda i, h: (0, h)),
            pl.BlockSpec((D, C), lambda i, h: (0, h)),
            pl.BlockSpec((1, 1, C), lambda i, h: (h, 0, 0)),
            pl.BlockSpec((1, 1, C), lambda i, h: (h, 0, 0)),
            pl.BlockSpec((1, 1, C), lambda i, h: (h, 0, 0)),
        ],
        out_specs=[
            pl.BlockSpec((1, RB, C), lambda i, h: (h, i, 0)),
            pl.BlockSpec((1, RB, C), lambda i, h: (h, i, 0)),
            pl.BlockSpec((1, RB, C), lambda i, h: (h, i, 0)),
        ],
        out_shape=[out_sh, out_sh, out_sh],
        compiler_params=pltpu.CompilerParams(
            dimension_semantics=("arbitrary", "arbitrary")),
    )(x_pad, Wq, Wk, Wv, bq.reshape(H, 1, C), bk.reshape(H, 1, C), bv.reshape(H, 1, C))


def _edge_body(q2, k2, v2, dst_hbm, src_hbm, agg_out, den_out,
               dstv, exv, qbuf, kbuf, gq, gk,
               red, redv, aggsh, densh, lmaxsh, sem, sem2):
    s = lax.axis_index("s")
    core = lax.axis_index("c")
    ebase = s * EPT
    lane = lax.iota(jnp.int32, 16)
    perms = [lane ^ sh for sh in (8, 4, 2, 1)]
    zero16 = jnp.zeros((16,), jnp.float32)
    neg16 = jnp.full((16,), -3e38, jnp.float32)

    def hsum(x):
        for p in perms:
            x = x + _shuf(x, p)
        return x

    def hmax(x):
        for p in perms:
            x = jnp.maximum(x, _shuf(x, p))
        return x

    pltpu.sync_copy(dst_hbm.at[pl.ds(s * NBLK, NBLK)], dstv)

    def head_step(j, _):
        hoff = (core * 4 + j) * NPAD

        def ze(t, _):
            exv[pl.ds(t * 16, 16)] = zero16
            return _
        lax.fori_loop(0, DROWS // 16, ze, None)
        pltpu.sync_copy(exv.at[pl.ds(0, DROWS)], densh.at[pl.ds(s * DROWS, DROWS)])

        def zk(r, _):
            for c8 in range(8):
                kbuf[r, pl.ds(c8 * 16, 16)] = zero16
            return _
        lax.fori_loop(0, BLK, zk, None)
        for t in range(AROWS // BLK):
            pltpu.sync_copy(kbuf, aggsh.at[pl.ds(s * AROWS + t * BLK, BLK)])
        REM = AROWS % BLK
        pltpu.sync_copy(kbuf.at[pl.ds(0, REM)],
                        aggsh.at[pl.ds(s * AROWS + (AROWS // BLK) * BLK, REM)])
        plsc.subcore_barrier()

        def s1(b, _):
            base = b * BLK
            pltpu.sync_copy(src_hbm.at[pl.ds(ebase + base, BLK)], gk)
            for t in range(BLK // 16):
                o = t * 16
                gq[pl.ds(o, 16)] = dstv[b, 0, pl.ds(o, 16)] + hoff
                gk[pl.ds(o, 16)] = gk[pl.ds(o, 16)] + hoff
            cp1 = pltpu.async_copy(q2.at[gq], qbuf, sem)
            cp2 = pltpu.async_copy(k2.at[gk], kbuf, sem2)
            cp1.wait()
            cp2.wait()
            for g in range(BLK // 16):

                def dot16(e, a16):
                    r = g * 16 + e
                    acc = qbuf[r, pl.ds(0, 16)] * kbuf[r, pl.ds(0, 16)]
                    for c8 in range(1, 8):
                        acc = acc + qbuf[r, pl.ds(c8 * 16, 16)] * kbuf[r, pl.ds(c8 * 16, 16)]
                    return jnp.where(lane == e, hsum(acc), a16)
                a16 = lax.fori_loop(0, 16, dot16, zero16)
                exv[pl.ds(base + g * 16, 16)] = a16 * INV_SQRT_C
            return _
        lax.fori_loop(0, NBLK, s1, None)

        def lm(t, m):
            return jnp.maximum(m, exv[pl.ds(t * 16, 16)])
        m = lax.fori_loop(0, EPT // 16, lm, neg16)
        redv[0, pl.ds(0, 16)] = hmax(m)
        pltpu.sync_copy(redv, lmaxsh.at[pl.ds(s, 1)])
        plsc.subcore_barrier()
        pltpu.sync_copy(lmaxsh, red)

        def gm(t, mm):
            return jnp.maximum(mm, red[t, pl.ds(0, 16)])
        gmax = hmax(lax.fori_loop(0, 16, gm, neg16))

        def s2(t, _):
            o = t * 16
            exv[pl.ds(o, 16)] = jnp.exp(exv[pl.ds(o, 16)] - gmax)
            return _
        lax.fori_loop(0, EPT // 16, s2, None)

        done = 0
        while done < NBLK:
            cnt = min(20, NBLK - done)
            cps = [pltpu.async_copy(
                exv.at[pl.ds((done + i) * BLK, BLK)],
                densh.at[dstv.at[done + i, 0]], sem, add=True)
                for i in range(cnt)]
            for cp in cps:
                cp.wait()
            done += cnt
        plsc.subcore_barrier()
        pltpu.sync_copy(densh.at[pl.ds(s * DROWS, DROWS)],
                        den_out.at[pl.ds(hoff + s * DROWS, DROWS)])

        def s3(b, _):
            base = b * BLK
            pltpu.sync_copy(src_hbm.at[pl.ds(ebase + base, BLK)], gk)
            for t in range(BLK // 16):
                o = t * 16
                gk[pl.ds(o, 16)] = gk[pl.ds(o, 16)] + hoff
            pltpu.async_copy(v2.at[gk], kbuf, sem).wait()
            for g in range(BLK // 16):
                e16 = exv[pl.ds(base + g * 16, 16)]

                def wmul(e, _):
                    w16 = _shuf(e16, jnp.full((16,), e, jnp.int32))
                    r = g * 16 + e
                    for c8 in range(8):
                        kbuf[r, pl.ds(c8 * 16, 16)] = kbuf[r, pl.ds(c8 * 16, 16)] * w16
                    return _
                lax.fori_loop(0, 16, wmul, None)
            pltpu.sync_copy(kbuf, aggsh.at[dstv.at[b, 0]], add=True)
            return _
        lax.fori_loop(0, NBLK, s3, None)
        plsc.subcore_barrier()

        pltpu.sync_copy(aggsh.at[pl.ds(s * AROWS, AROWS)],
                        agg_out.at[pl.ds(hoff + s * AROWS, AROWS)])
        plsc.subcore_barrier()
        return _

    lax.fori_loop(0, 4, head_step, None)


def _edge_phase(q2, k2, v2, dst_p3, src_p):
    mesh = plsc.VectorSubcoreMesh(core_axis_name="c", subcore_axis_name="s")
    return pl.kernel(
        _edge_body,
        out_type=[jax.ShapeDtypeStruct((H * NPAD, C), jnp.float32),
                  jax.ShapeDtypeStruct((H * NPAD,), jnp.float32)],
        mesh=mesh,
        scratch_types=[
            pltpu.VMEM((NBLK, 1, BLK), jnp.int32),
            pltpu.VMEM((EPT,), jnp.float32),
            pltpu.VMEM((BLK, C), jnp.float32),
            pltpu.VMEM((BLK, C), jnp.float32),
            pltpu.VMEM((BLK,), jnp.int32),
            pltpu.VMEM((BLK,), jnp.int32),
            pltpu.VMEM((16, 16), jnp.float32),
            pltpu.VMEM((1, 16), jnp.float32),
            pltpu.VMEM_SHARED((NSC, C), jnp.float32),
            pltpu.VMEM_SHARED((NPAD,), jnp.float32),
            pltpu.VMEM_SHARED((16, 16), jnp.float32),
            pltpu.SemaphoreType.DMA,
            pltpu.SemaphoreType.DMA,
        ],
    )(q2, k2, v2, dst_p3, src_p)


def _tail_body(x_ref, agg_ref, den_ref, wskip_ref, bskip_ref, g1_ref, b1_ref,
               w1_ref, bf1_ref, w2_ref, bf2_ref, g2_ref, b2_ref, o_ref):
    xb = x_ref[...]
    aggm = jnp.mean(agg_ref[...] / (den_ref[...][:, :, None] + 1e-16), axis=0)
    attn = aggm + jnp.dot(xb, wskip_ref[...], preferred_element_type=jnp.float32) + bskip_ref[0]
    t = xb + attn
    mu = jnp.mean(t, axis=-1, keepdims=True)
    var = jnp.mean((t - mu) ** 2, axis=-1, keepdims=True)
    h1 = (t - mu) * lax.rsqrt(var + 1e-5) * g1_ref[0] + b1_ref[0]
    ff = jnp.maximum(jnp.dot(h1, w1_ref[...], preferred_element_type=jnp.float32) + bf1_ref[0], 0.0)
    ff = jnp.dot(ff, w2_ref[...], preferred_element_type=jnp.float32) + bf2_ref[0]
    t2 = h1 + ff
    mu2 = jnp.mean(t2, axis=-1, keepdims=True)
    var2 = jnp.mean((t2 - mu2) ** 2, axis=-1, keepdims=True)
    o_ref[...] = (t2 - mu2) * lax.rsqrt(var2 + 1e-5) * g2_ref[0] + b2_ref[0]


def _tail(x_pad, agg, den, Wskip, bskip, ln1_g, ln1_b, W1, b1, W2, b2, ln2_g, ln2_b):
    RB = 512
    grid = (NPAD // RB,)
    r1 = lambda a: a.reshape(1, -1)
    return pl.pallas_call(
        _tail_body,
        grid=grid,
        in_specs=[
            pl.BlockSpec((RB, D), lambda i: (i, 0)),
            pl.BlockSpec((H, RB, C), lambda i: (0, i, 0)),
            pl.BlockSpec((H, RB), lambda i: (0, i)),
            pl.BlockSpec((D, D), lambda i: (0, 0)),
            pl.BlockSpec((1, D), lambda i: (0, 0)),
            pl.BlockSpec((1, D), lambda i: (0, 0)),
            pl.BlockSpec((1, D), lambda i: (0, 0)),
            pl.BlockSpec((D, FF), lambda i: (0, 0)),
            pl.BlockSpec((1, FF), lambda i: (0, 0)),
            pl.BlockSpec((FF, D), lambda i: (0, 0)),
            pl.BlockSpec((1, D), lambda i: (0, 0)),
            pl.BlockSpec((1, D), lambda i: (0, 0)),
            pl.BlockSpec((1, D), lambda i: (0, 0)),
        ],
        out_specs=pl.BlockSpec((RB, D), lambda i: (i, 0)),
        out_shape=jax.ShapeDtypeStruct((NPAD, D), jnp.float32),
        compiler_params=pltpu.CompilerParams(
            dimension_semantics=("arbitrary",)),
    )(x_pad, agg, den, Wskip, r1(bskip), r1(ln1_g), r1(ln1_b),
      W1, r1(b1), W2, r1(b2), r1(ln2_g), r1(ln2_b))


def kernel(x, edge_index, Wq, bq, Wk, bk, Wv, bv, Wskip, bskip,
           ln1_g, ln1_b, W1, b1, W2, b2, ln2_g, ln2_b):
    x_pad = jnp.pad(x, ((0, NPAD - N), (0, 0)))
    src = edge_index[0]
    dst = edge_index[1]
    pad_ar = jnp.arange(EPAD - E, dtype=jnp.int32)
    dst_p = jnp.concatenate([dst, N + pad_ar % (NSC - N)])
    src_p = jnp.concatenate([src, pad_ar % N])

    q2, k2, v2 = _qkv(x_pad, Wq, Wk, Wv, bq, bk, bv)
    agg, den = _edge_phase(q2.reshape(H * NPAD, C), k2.reshape(H * NPAD, C),
                           v2.reshape(H * NPAD, C),
                           dst_p.reshape(EPAD // BLK, 1, BLK), src_p)
    out = _tail(x_pad, agg.reshape(H, NPAD, C), den.reshape(H, NPAD),
                Wskip, bskip, ln1_g, ln1_b, W1, b1, W2, b2, ln2_g, ln2_b)
    return out[:N]
